# Initial kernel scaffold; baseline (speedup 1.0000x reference)
import jax
import jax.numpy as jnp
from jax.experimental import pallas as pl


def kernel(x, edge_index, batch, W1, b1, W2, b2, W3, b3, Wl, bl):
    raise NotImplementedError("write your pallas kernel here")


# speedup vs baseline: 1.8418x; 1.8418x over previous
import functools

import jax
import jax.numpy as jnp
from jax import lax
from jax.experimental import pallas as pl
from jax.experimental.pallas import tpu as pltpu
from jax.experimental.pallas import tpu_sc as plsc

N = 10000
E = 320000
D_IN = 128
H = 64
C = 2
G = 64

NC = 2
NS = 16
NW = NC * NS
EPW = E // NW
K = 80
CH = EPW // K
L = 16
NPAD = 10240
NR = NPAD // 2
RPW = NR // NS
HP = 128
R = 1000


def _mesh():
    return plsc.VectorSubcoreMesh(core_axis_name="c", subcore_axis_name="s")


def _prep(src, dst, ones2, zeros_acc):
    @functools.partial(
        pl.kernel,
        out_type=(
            jax.ShapeDtypeStruct((NC, NR, HP), jnp.float32),
            jax.ShapeDtypeStruct((NW, EPW), jnp.int32),
            jax.ShapeDtypeStruct((NW, CH, K), jnp.int32),
        ),
        mesh=_mesh(),
        scratch_types=[
            pltpu.VMEM((EPW,), jnp.int32),
            pltpu.VMEM((CH, K), jnp.int32),
            pltpu.VMEM((EPW,), jnp.int32),
            pltpu.VMEM((K, HP), jnp.float32),
            pltpu.VMEM_SHARED((NR, HP), jnp.float32),
        ],
    )
    def k(src_hbm, dst_hbm, ones_hbm, z_hbm, deg_hbm, gi_hbm, ai_hbm,
          sg, da, pi, rows, acc):
        c = lax.axis_index("c")
        s = lax.axis_index("s")
        w = c * NS + s
        pltpu.sync_copy(z_hbm.at[pl.ds(s * RPW, RPW)],
                        acc.at[pl.ds(s * RPW, RPW)])
        pltpu.sync_copy(src_hbm.at[w], sg)
        pltpu.sync_copy(dst_hbm.at[w], da)

        @pl.loop(0, CH)
        def _(r):
            for q in range(K // L):
                sl2 = pl.ds(q * L, L)
                sl1 = pl.ds(r * K + q * L, L)
                sv = sg[sl1]
                dv = da[r, sl2]
                par = lax.bitwise_and(dv, 1)
                sg[sl1] = sv * 2 + par
                da[r, sl2] = lax.shift_right_logical(dv, 1)
                pi[sl1] = par

        pltpu.sync_copy(sg, gi_hbm.at[w])
        pltpu.sync_copy(da, ai_hbm.at[w])
        plsc.subcore_barrier()

        @pl.loop(0, CH)
        def _(j):
            pltpu.sync_copy(ones_hbm.at[pi.at[pl.ds(j * K, K)]], rows)
            pltpu.sync_copy(rows, acc.at[da.at[j]], add=True)

        plsc.subcore_barrier()
        pltpu.sync_copy(acc.at[pl.ds(s * RPW, RPW)],
                        deg_hbm.at[c].at[pl.ds(s * RPW, RPW)])

    return k(src, dst, ones2, zeros_acc)


def _scat(g2, gidx, aidx, zeros_acc):
    @functools.partial(
        pl.kernel,
        out_type=jax.ShapeDtypeStruct((NC, NR, HP), jnp.float32),
        mesh=_mesh(),
        scratch_types=[
            pltpu.VMEM((EPW,), jnp.int32),
            pltpu.VMEM((CH, K), jnp.int32),
            pltpu.VMEM((K, HP), jnp.float32),
            pltpu.VMEM((K, HP), jnp.float32),
            pltpu.VMEM_SHARED((NR, HP), jnp.float32),
            pltpu.SemaphoreType.DMA,
            pltpu.SemaphoreType.DMA,
        ],
    )
    def k(g_hbm, gi_hbm, ai_hbm, z_hbm, out_hbm,
          gi, ai, r0, r1, acc, sem0, sem1):
        c = lax.axis_index("c")
        s = lax.axis_index("s")
        w = c * NS + s
        pltpu.sync_copy(z_hbm.at[pl.ds(s * RPW, RPW)],
                        acc.at[pl.ds(s * RPW, RPW)])
        pltpu.sync_copy(gi_hbm.at[w], gi)
        pltpu.sync_copy(ai_hbm.at[w], ai)
        plsc.subcore_barrier()

        pltpu.async_copy(g_hbm.at[gi.at[pl.ds(0, K)]], r0, sem0)

        @pl.loop(0, CH // 2)
        def _(i):
            j = 2 * i
            pltpu.make_async_copy(g_hbm.at[gi.at[pl.ds(j * K, K)]], r0, sem0).wait()
            pltpu.async_copy(g_hbm.at[gi.at[pl.ds((j + 1) * K, K)]], r1, sem1)
            pltpu.sync_copy(r0, acc.at[ai.at[j]], add=True)

            @pl.when(j + 2 < CH)
            def _():
                pltpu.async_copy(g_hbm.at[gi.at[pl.ds((j + 2) * K, K)]], r0, sem0)

            pltpu.make_async_copy(g_hbm.at[gi.at[pl.ds((j + 1) * K, K)]], r1, sem1).wait()
            pltpu.sync_copy(r1, acc.at[ai.at[j + 1]], add=True)

        pltpu.make_async_copy(g_hbm.at[gi.at[pl.ds((CH - 1) * K, K)]], r0, sem0).wait()
        pltpu.sync_copy(r0, acc.at[ai.at[CH - 1]], add=True)

        plsc.subcore_barrier()
        pltpu.sync_copy(acc.at[pl.ds(s * RPW, RPW)],
                        out_hbm.at[c].at[pl.ds(s * RPW, RPW)])

    return k(g2, gidx, aidx, zeros_acc)


def _dinv_of(d_ref):
    deg = d_ref[0][:, 0:1] + d_ref[1][:, 0:1] + 1.0
    return lax.rsqrt(deg)


def _pack(t):
    z = jnp.zeros((R, 2 * H), jnp.float32)
    return jnp.concatenate([t, z, t], axis=1)


def _mm_first(x, W1, degp):
    def body(x_ref, w_ref, d_ref, o_ref):
        dinv = _dinv_of(d_ref)
        t = jnp.dot(x_ref[...], w_ref[...],
                    preferred_element_type=jnp.float32,
                    precision=lax.Precision.HIGHEST)
        o_ref[...] = _pack(t * dinv)

    return pl.pallas_call(
        body,
        grid=(N // R,),
        in_specs=[
            pl.BlockSpec((R, D_IN), lambda i: (i, 0)),
            pl.BlockSpec((D_IN, H), lambda i: (0, 0)),
            pl.BlockSpec((NC, R, H), lambda i: (0, i, 0)),
        ],
        out_specs=pl.BlockSpec((R, 4 * H), lambda i: (i, 0)),
        out_shape=jax.ShapeDtypeStruct((N, 4 * H), jnp.float32),
    )(x, W1, degp)


def _mm_mid(p, g_prev, degp, b, W):
    def body(p_ref, g_ref, d_ref, b_ref, w_ref, o_ref):
        dinv = _dinv_of(d_ref)
        h = dinv * (p_ref[0] + p_ref[1] + g_ref[:, :H]) + b_ref[...]
        h = jnp.maximum(h, 0.0)
        t = jnp.dot(h, w_ref[...],
                    preferred_element_type=jnp.float32,
                    precision=lax.Precision.HIGHEST) * dinv
        o_ref[...] = _pack(t)

    return pl.pallas_call(
        body,
        grid=(N // R,),
        in_specs=[
            pl.BlockSpec((NC, R, H), lambda i: (0, i, 0)),
            pl.BlockSpec((R, 4 * H), lambda i: (i, 0)),
            pl.BlockSpec((NC, R, H), lambda i: (0, i, 0)),
            pl.BlockSpec((1, H), lambda i: (0, 0)),
            pl.BlockSpec((H, H), lambda i: (0, 0)),
        ],
        out_specs=pl.BlockSpec((R, 4 * H), lambda i: (i, 0)),
        out_shape=jax.ShapeDtypeStruct((N, 4 * H), jnp.float32),
    )(p, g_prev, degp, b.reshape(1, H), W)


def _final(p, g_prev, degp, b, batch3, Wl, bl):
    def body(p_ref, g_ref, d_ref, b_ref, bat_ref, wl_ref, bl_ref,
             o_ref, acc_ref):
        i = pl.program_id(0)

        @pl.when(i == 0)
        def _():
            acc_ref[...] = jnp.zeros_like(acc_ref)

        dinv = _dinv_of(d_ref)
        h = dinv * (p_ref[0] + p_ref[1] + g_ref[:, :H]) + b_ref[...]
        bat = bat_ref[0]
        gid = lax.broadcasted_iota(jnp.int32, (G, R), 0)
        m = (gid == bat).astype(jnp.float32)
        haug = jnp.concatenate(
            [h, jnp.ones((R, H), jnp.float32)], axis=1)
        acc_ref[...] += jnp.dot(m, haug,
                                preferred_element_type=jnp.float32,
                                precision=lax.Precision.HIGHEST)

        @pl.when(i == N // R - 1)
        def _():
            sums = acc_ref[:, :H]
            cnt = acc_ref[:, H:]
            pooled = sums / jnp.maximum(cnt, 1.0)
            o_ref[...] = jnp.dot(pooled, wl_ref[...],
                                 preferred_element_type=jnp.float32,
                                 precision=lax.Precision.HIGHEST) + bl_ref[...]

    return pl.pallas_call(
        body,
        grid=(N // R,),
        in_specs=[
            pl.BlockSpec((NC, R, H), lambda i: (0, i, 0)),
            pl.BlockSpec((R, 4 * H), lambda i: (i, 0)),
            pl.BlockSpec((NC, R, H), lambda i: (0, i, 0)),
            pl.BlockSpec((1, H), lambda i: (0, 0)),
            pl.BlockSpec((1, 1, R), lambda i: (i, 0, 0)),
            pl.BlockSpec((H, C), lambda i: (0, 0)),
            pl.BlockSpec((1, C), lambda i: (0, 0)),
        ],
        out_specs=pl.BlockSpec((G, C), lambda i: (0, 0)),
        out_shape=jax.ShapeDtypeStruct((G, C), jnp.float32),
        scratch_shapes=[pltpu.VMEM((G, 2 * H), jnp.float32)],
    )(p, g_prev, degp, b.reshape(1, H), batch3, Wl, bl.reshape(1, C))


def kernel(x, edge_index, batch, W1, b1, W2, b2, W3, b3, Wl, bl):
    src = edge_index[0].reshape(NW, EPW)
    dst = edge_index[1].reshape(NW, CH, K)
    batch3 = batch.reshape(N // R, 1, R)

    zeros_acc = jnp.zeros((NR, HP), jnp.float32)
    ones2 = jnp.repeat(jnp.eye(2, dtype=jnp.float32), H, axis=1)

    degp3, gidx, aidx = _prep(src, dst, ones2, zeros_acc)
    degp = degp3.reshape(NC, NPAD, H)

    g1 = _mm_first(x, W1, degp)
    p1 = _scat(g1.reshape(2 * N, HP), gidx, aidx, zeros_acc)
    g2 = _mm_mid(p1.reshape(NC, NPAD, H), g1, degp, b1, W2)
    p2 = _scat(g2.reshape(2 * N, HP), gidx, aidx, zeros_acc)
    g3 = _mm_mid(p2.reshape(NC, NPAD, H), g2, degp, b2, W3)
    p3 = _scat(g3.reshape(2 * N, HP), gidx, aidx, zeros_acc)
    return _final(p3.reshape(NC, NPAD, H), g3, degp, b3, batch3, Wl, bl)


# speedup vs baseline: 15.2472x; 8.2785x over previous
import dataclasses
import functools

import jax
import jax.numpy as jnp
from jax import lax
from jax.experimental import pallas as pl
from jax.experimental.pallas import tpu as pltpu
from jax.experimental.pallas import tpu_sc as plsc

N = 10000
E = 320000
D_IN = 128
H = 64
C = 2
G = 64

NC = 2
NS = 16
NW = NC * NS
EPW = E // NW
K = 80
CH = EPW // K
L = 16
NPAD = 10240
NR = NPAD // 2
RPW = NR // NS
HP = 128
NHALF = N // 2
R = 1000


def _mesh():
    return plsc.VectorSubcoreMesh(core_axis_name="c", subcore_axis_name="s")


def _sc_params():
    cp = pltpu.CompilerParams()
    if "needs_layout_passes" in pltpu.CompilerParams.__dataclass_fields__:
        cp = dataclasses.replace(cp, needs_layout_passes=False)
    return cp


def _prep(src, dst, zeros_h):
    @functools.partial(
        pl.kernel,
        out_type=(
            jax.ShapeDtypeStruct((NW, 2, NHALF * L), jnp.float32),
            jax.ShapeDtypeStruct((NW, EPW), jnp.int32),
            jax.ShapeDtypeStruct((NW, CH, K), jnp.int32),
        ),
        mesh=_mesh(),
        compiler_params=_sc_params(),
        scratch_types=[
            pltpu.VMEM((EPW,), jnp.int32),
            pltpu.VMEM((CH, K), jnp.int32),
            pltpu.VMEM((NHALF * L,), jnp.float32),
        ],
    )
    def k(src_hbm, dst_hbm, zh_hbm, hist_hbm, gi_hbm, ai_hbm, sg, da, hist):
        c = lax.axis_index("c")
        s = lax.axis_index("s")
        w = c * NS + s
        pltpu.sync_copy(src_hbm.at[w], sg)
        pltpu.sync_copy(dst_hbm.at[w], da)
        lanes = lax.iota(jnp.int32, L)
        ones_v = jnp.ones((L,), jnp.float32)

        for half in range(2):
            lo = half * NHALF
            pltpu.sync_copy(zh_hbm, hist)

            @pl.loop(0, CH)
            def _(r):
                for q in range(K // L):
                    dv = da[r, pl.ds(q * L, L)]
                    loc = dv - lo
                    msk = jnp.logical_and(loc >= 0, loc < NHALF)
                    plsc.addupdate_scatter(
                        hist, [loc * L + lanes], ones_v, mask=msk)

            pltpu.sync_copy(hist, hist_hbm.at[w, half])

        @pl.loop(0, CH)
        def _(r):
            for q in range(K // L):
                sl2 = pl.ds(q * L, L)
                sl1 = pl.ds(r * K + q * L, L)
                sv = sg[sl1]
                dv = da[r, sl2]
                par = lax.bitwise_and(dv, 1)
                sg[sl1] = sv * 2 + par
                da[r, sl2] = lax.shift_right_logical(dv, 1)

        pltpu.sync_copy(sg, gi_hbm.at[w])
        pltpu.sync_copy(da, ai_hbm.at[w])

    return k(src, dst, zeros_h)


def _scat(g2, gidx, aidx, zeros_acc):
    @functools.partial(
        pl.kernel,
        out_type=jax.ShapeDtypeStruct((NC, NR, HP), jnp.float32),
        mesh=_mesh(),
        scratch_types=[
            pltpu.VMEM((EPW,), jnp.int32),
            pltpu.VMEM((CH, K), jnp.int32),
            pltpu.VMEM((K, HP), jnp.float32),
            pltpu.VMEM((K, HP), jnp.float32),
            pltpu.VMEM_SHARED((NR, HP), jnp.float32),
            pltpu.SemaphoreType.DMA,
            pltpu.SemaphoreType.DMA,
        ],
    )
    def k(g_hbm, gi_hbm, ai_hbm, z_hbm, out_hbm,
          gi, ai, r0, r1, acc, sem0, sem1):
        c = lax.axis_index("c")
        s = lax.axis_index("s")
        w = c * NS + s
        pltpu.sync_copy(z_hbm.at[pl.ds(s * RPW, RPW)],
                        acc.at[pl.ds(s * RPW, RPW)])
        pltpu.sync_copy(gi_hbm.at[w], gi)
        pltpu.sync_copy(ai_hbm.at[w], ai)
        plsc.subcore_barrier()

        pltpu.async_copy(g_hbm.at[gi.at[pl.ds(0, K)]], r0, sem0)

        @pl.loop(0, CH // 2)
        def _(i):
            j = 2 * i
            pltpu.make_async_copy(g_hbm.at[gi.at[pl.ds(j * K, K)]], r0, sem0).wait()
            pltpu.async_copy(g_hbm.at[gi.at[pl.ds((j + 1) * K, K)]], r1, sem1)
            pltpu.sync_copy(r0, acc.at[ai.at[j]], add=True)

            @pl.when(j + 2 < CH)
            def _():
                pltpu.async_copy(g_hbm.at[gi.at[pl.ds((j + 2) * K, K)]], r0, sem0)

            pltpu.make_async_copy(g_hbm.at[gi.at[pl.ds((j + 1) * K, K)]], r1, sem1).wait()
            pltpu.sync_copy(r1, acc.at[ai.at[j + 1]], add=True)

        pltpu.make_async_copy(g_hbm.at[gi.at[pl.ds((CH - 1) * K, K)]], r0, sem0).wait()
        pltpu.sync_copy(r0, acc.at[ai.at[CH - 1]], add=True)

        plsc.subcore_barrier()
        pltpu.sync_copy(acc.at[pl.ds(s * RPW, RPW)],
                        out_hbm.at[c].at[pl.ds(s * RPW, RPW)])

    return k(g2, gidx, aidx, zeros_acc)


def _degsum(histr):
    def body(h_ref, o_ref):
        deg = jnp.sum(h_ref[...], axis=(0, 2)) + 1.0
        o_ref[...] = jnp.broadcast_to(lax.rsqrt(deg)[:, None], (R, 8))

    return pl.pallas_call(
        body,
        grid=(N // R,),
        in_specs=[pl.BlockSpec((NW, R, L), lambda i: (0, i, 0))],
        out_specs=pl.BlockSpec((R, 8), lambda i: (i, 0)),
        out_shape=jax.ShapeDtypeStruct((N, 8), jnp.float32),
    )(histr)


def _pack(t):
    z = jnp.zeros((R, 2 * H), jnp.float32)
    return jnp.concatenate([t, z, t], axis=1)


def _mm_first(x, W1, dinv):
    def body(x_ref, w_ref, d_ref, o_ref):
        dinv = d_ref[:, 0:1]
        t = jnp.dot(x_ref[...], w_ref[...],
                    preferred_element_type=jnp.float32,
                    precision=lax.Precision.HIGHEST)
        o_ref[...] = _pack(t * dinv)

    return pl.pallas_call(
        body,
        grid=(N // R,),
        in_specs=[
            pl.BlockSpec((R, D_IN), lambda i: (i, 0)),
            pl.BlockSpec((D_IN, H), lambda i: (0, 0)),
            pl.BlockSpec((R, 8), lambda i: (i, 0)),
        ],
        out_specs=pl.BlockSpec((R, 4 * H), lambda i: (i, 0)),
        out_shape=jax.ShapeDtypeStruct((N, 4 * H), jnp.float32),
    )(x, W1, dinv)


def _mm_mid(p, g_prev, dinv, b, W):
    def body(p_ref, g_ref, d_ref, b_ref, w_ref, o_ref):
        dinv = d_ref[:, 0:1]
        h = dinv * (p_ref[0] + p_ref[1] + g_ref[:, :H]) + b_ref[...]
        h = jnp.maximum(h, 0.0)
        t = jnp.dot(h, w_ref[...],
                    preferred_element_type=jnp.float32,
                    precision=lax.Precision.HIGHEST) * dinv
        o_ref[...] = _pack(t)

    return pl.pallas_call(
        body,
        grid=(N // R,),
        in_specs=[
            pl.BlockSpec((NC, R, H), lambda i: (0, i, 0)),
            pl.BlockSpec((R, 4 * H), lambda i: (i, 0)),
            pl.BlockSpec((R, 8), lambda i: (i, 0)),
            pl.BlockSpec((1, H), lambda i: (0, 0)),
            pl.BlockSpec((H, H), lambda i: (0, 0)),
        ],
        out_specs=pl.BlockSpec((R, 4 * H), lambda i: (i, 0)),
        out_shape=jax.ShapeDtypeStruct((N, 4 * H), jnp.float32),
    )(p, g_prev, dinv, b.reshape(1, H), W)


def _final(p, g_prev, dinv, b, batch3, Wl, bl):
    def body(p_ref, g_ref, d_ref, b_ref, bat_ref, wl_ref, bl_ref,
             o_ref, acc_ref):
        i = pl.program_id(0)

        @pl.when(i == 0)
        def _():
            acc_ref[...] = jnp.zeros_like(acc_ref)

        dinv = d_ref[:, 0:1]
        h = dinv * (p_ref[0] + p_ref[1] + g_ref[:, :H]) + b_ref[...]
        bat = bat_ref[0]
        gid = lax.broadcasted_iota(jnp.int32, (G, R), 0)
        m = (gid == bat).astype(jnp.float32)
        haug = jnp.concatenate(
            [h, jnp.ones((R, H), jnp.float32)], axis=1)
        acc_ref[...] += jnp.dot(m, haug,
                                preferred_element_type=jnp.float32,
                                precision=lax.Precision.HIGHEST)

        @pl.when(i == N // R - 1)
        def _():
            sums = acc_ref[:, :H]
            cnt = acc_ref[:, H:]
            pooled = sums / jnp.maximum(cnt, 1.0)
            o_ref[...] = jnp.dot(pooled, wl_ref[...],
                                 preferred_element_type=jnp.float32,
                                 precision=lax.Precision.HIGHEST) + bl_ref[...]

    return pl.pallas_call(
        body,
        grid=(N // R,),
        in_specs=[
            pl.BlockSpec((NC, R, H), lambda i: (0, i, 0)),
            pl.BlockSpec((R, 4 * H), lambda i: (i, 0)),
            pl.BlockSpec((R, 8), lambda i: (i, 0)),
            pl.BlockSpec((1, H), lambda i: (0, 0)),
            pl.BlockSpec((1, 1, R), lambda i: (i, 0, 0)),
            pl.BlockSpec((H, C), lambda i: (0, 0)),
            pl.BlockSpec((1, C), lambda i: (0, 0)),
        ],
        out_specs=pl.BlockSpec((G, C), lambda i: (0, 0)),
        out_shape=jax.ShapeDtypeStruct((G, C), jnp.float32),
        scratch_shapes=[pltpu.VMEM((G, 2 * H), jnp.float32)],
    )(p, g_prev, dinv, b.reshape(1, H), batch3, Wl, bl.reshape(1, C))


def kernel(x, edge_index, batch, W1, b1, W2, b2, W3, b3, Wl, bl):
    src = edge_index[0].reshape(NW, EPW)
    dst = edge_index[1].reshape(NW, CH, K)
    batch3 = batch.reshape(N // R, 1, R)

    zeros_acc = jnp.zeros((NR, HP), jnp.float32)
    zeros_h = jnp.zeros((NHALF * L,), jnp.float32)

    histp, gidx, aidx = _prep(src, dst, zeros_h)
    dinv = _degsum(histp.reshape(NW, N, L))

    g1 = _mm_first(x, W1, dinv)
    p1 = _scat(g1.reshape(2 * N, HP), gidx, aidx, zeros_acc)
    g2 = _mm_mid(p1.reshape(NC, NPAD, H), g1, dinv, b1, W2)
    p2 = _scat(g2.reshape(2 * N, HP), gidx, aidx, zeros_acc)
    g3 = _mm_mid(p2.reshape(NC, NPAD, H), g2, dinv, b2, W3)
    p3 = _scat(g3.reshape(2 * N, HP), gidx, aidx, zeros_acc)
    return _final(p3.reshape(NC, NPAD, H), g3, dinv, b3, batch3, Wl, bl)


# speedup vs baseline: 21.7313x; 1.4253x over previous
import dataclasses
import functools

import jax
import jax.numpy as jnp
from jax import lax
from jax.experimental import pallas as pl
from jax.experimental.pallas import tpu as pltpu
from jax.experimental.pallas import tpu_sc as plsc

N = 10000
E = 320000
D_IN = 128
H = 64
C = 2
G = 64

NC = 2
NS = 16
NW = NC * NS
EPW = E // NW
K = 80
CH = EPW // K
L = 16
NPAD = 10240
NR = NPAD // 2
RPW = NR // NS
HP = 128
NHALF = N // 2
R = 1000


def _mesh():
    return plsc.VectorSubcoreMesh(core_axis_name="c", subcore_axis_name="s")


def _sc_params():
    cp = pltpu.CompilerParams()
    if "needs_layout_passes" in pltpu.CompilerParams.__dataclass_fields__:
        cp = dataclasses.replace(cp, needs_layout_passes=False)
    return cp


def _prep(src, dst, zeros_h):
    @functools.partial(
        pl.kernel,
        out_type=(
            jax.ShapeDtypeStruct((NW, 2, L, NHALF), jnp.float32),
            jax.ShapeDtypeStruct((NW, EPW), jnp.int32),
            jax.ShapeDtypeStruct((NW, CH, K), jnp.int32),
        ),
        mesh=_mesh(),
        compiler_params=_sc_params(),
        scratch_types=[
            pltpu.VMEM((EPW,), jnp.int32),
            pltpu.VMEM((CH, K), jnp.int32),
            pltpu.VMEM((L, NHALF), jnp.float32),
        ],
    )
    def k(src_hbm, dst_hbm, zh_hbm, hist_hbm, gi_hbm, ai_hbm, sg, da, hist):
        c = lax.axis_index("c")
        s = lax.axis_index("s")
        w = c * NS + s
        pltpu.sync_copy(src_hbm.at[w], sg)
        pltpu.sync_copy(dst_hbm.at[w], da)
        lanes = lax.iota(jnp.int32, L)
        ones_v = jnp.ones((L,), jnp.float32)

        for half in range(2):
            lo = half * NHALF
            pltpu.sync_copy(zh_hbm, hist)

            @pl.loop(0, CH)
            def _(r):
                for q in range(K // L):
                    dv = da[r, pl.ds(q * L, L)]
                    loc = dv - lo
                    msk = jnp.logical_and(loc >= 0, loc < NHALF)
                    plsc.addupdate_scatter(
                        hist, [lanes, loc], ones_v, mask=msk)

            pltpu.sync_copy(hist, hist_hbm.at[w, half])

        @pl.loop(0, CH)
        def _(r):
            for q in range(K // L):
                sl2 = pl.ds(q * L, L)
                sl1 = pl.ds(r * K + q * L, L)
                sv = sg[sl1]
                dv = da[r, sl2]
                par = lax.bitwise_and(dv, 1)
                sg[sl1] = sv * 2 + par
                da[r, sl2] = lax.shift_right_logical(dv, 1)

        pltpu.sync_copy(sg, gi_hbm.at[w])
        pltpu.sync_copy(da, ai_hbm.at[w])

    return k(src, dst, zeros_h)


def _scat(g2, gidx, aidx, zeros_acc):
    @functools.partial(
        pl.kernel,
        out_type=jax.ShapeDtypeStruct((NC, NR, HP), jnp.float32),
        mesh=_mesh(),
        scratch_types=[
            pltpu.VMEM((EPW,), jnp.int32),
            pltpu.VMEM((CH, K), jnp.int32),
            pltpu.VMEM((K, HP), jnp.float32),
            pltpu.VMEM((K, HP), jnp.float32),
            pltpu.VMEM_SHARED((NR, HP), jnp.float32),
            pltpu.SemaphoreType.DMA,
            pltpu.SemaphoreType.DMA,
        ],
    )
    def k(g_hbm, gi_hbm, ai_hbm, z_hbm, out_hbm,
          gi, ai, r0, r1, acc, sem0, sem1):
        c = lax.axis_index("c")
        s = lax.axis_index("s")
        w = c * NS + s
        pltpu.sync_copy(z_hbm.at[pl.ds(s * RPW, RPW)],
                        acc.at[pl.ds(s * RPW, RPW)])
        pltpu.sync_copy(gi_hbm.at[w], gi)
        pltpu.sync_copy(ai_hbm.at[w], ai)
        plsc.subcore_barrier()

        pltpu.async_copy(g_hbm.at[gi.at[pl.ds(0, K)]], r0, sem0)

        @pl.loop(0, CH // 2)
        def _(i):
            j = 2 * i
            pltpu.make_async_copy(g_hbm.at[gi.at[pl.ds(j * K, K)]], r0, sem0).wait()
            pltpu.async_copy(g_hbm.at[gi.at[pl.ds((j + 1) * K, K)]], r1, sem1)
            pltpu.sync_copy(r0, acc.at[ai.at[j]], add=True)

            @pl.when(j + 2 < CH)
            def _():
                pltpu.async_copy(g_hbm.at[gi.at[pl.ds((j + 2) * K, K)]], r0, sem0)

            pltpu.make_async_copy(g_hbm.at[gi.at[pl.ds((j + 1) * K, K)]], r1, sem1).wait()
            pltpu.sync_copy(r1, acc.at[ai.at[j + 1]], add=True)

        pltpu.make_async_copy(g_hbm.at[gi.at[pl.ds((CH - 1) * K, K)]], r0, sem0).wait()
        pltpu.sync_copy(r0, acc.at[ai.at[CH - 1]], add=True)

        plsc.subcore_barrier()
        pltpu.sync_copy(acc.at[pl.ds(s * RPW, RPW)],
                        out_hbm.at[c].at[pl.ds(s * RPW, RPW)])

    return k(g2, gidx, aidx, zeros_acc)


def _degsum(histr):
    def body(h_ref, o_ref):
        deg = jnp.sum(h_ref[...], axis=(0, 1, 2)) + 1.0
        o_ref[...] = jnp.broadcast_to(lax.rsqrt(deg)[:, None], (NHALF, 8))

    return pl.pallas_call(
        body,
        grid=(2,),
        in_specs=[pl.BlockSpec((NW, 1, L, NHALF), lambda i: (0, i, 0, 0))],
        out_specs=pl.BlockSpec((NHALF, 8), lambda i: (i, 0)),
        out_shape=jax.ShapeDtypeStruct((N, 8), jnp.float32),
    )(histr)


def _pack(t):
    z = jnp.zeros((R, 2 * H), jnp.float32)
    return jnp.concatenate([t, z, t], axis=1)


def _mm_first(x, W1, dinv):
    def body(x_ref, w_ref, d_ref, o_ref):
        dinv = d_ref[:, 0:1]
        t = jnp.dot(x_ref[...], w_ref[...],
                    preferred_element_type=jnp.float32,
                    precision=lax.Precision.HIGHEST)
        o_ref[...] = _pack(t * dinv)

    return pl.pallas_call(
        body,
        grid=(N // R,),
        in_specs=[
            pl.BlockSpec((R, D_IN), lambda i: (i, 0)),
            pl.BlockSpec((D_IN, H), lambda i: (0, 0)),
            pl.BlockSpec((R, 8), lambda i: (i, 0)),
        ],
        out_specs=pl.BlockSpec((R, 4 * H), lambda i: (i, 0)),
        out_shape=jax.ShapeDtypeStruct((N, 4 * H), jnp.float32),
    )(x, W1, dinv)


def _mm_mid(p, g_prev, dinv, b, W):
    def body(p_ref, g_ref, d_ref, b_ref, w_ref, o_ref):
        dinv = d_ref[:, 0:1]
        h = dinv * (p_ref[0] + p_ref[1] + g_ref[:, :H]) + b_ref[...]
        h = jnp.maximum(h, 0.0)
        t = jnp.dot(h, w_ref[...],
                    preferred_element_type=jnp.float32,
                    precision=lax.Precision.HIGHEST) * dinv
        o_ref[...] = _pack(t)

    return pl.pallas_call(
        body,
        grid=(N // R,),
        in_specs=[
            pl.BlockSpec((NC, R, H), lambda i: (0, i, 0)),
            pl.BlockSpec((R, 2 * H), lambda i: (i, 0)),
            pl.BlockSpec((R, 8), lambda i: (i, 0)),
            pl.BlockSpec((1, H), lambda i: (0, 0)),
            pl.BlockSpec((H, H), lambda i: (0, 0)),
        ],
        out_specs=pl.BlockSpec((R, 4 * H), lambda i: (i, 0)),
        out_shape=jax.ShapeDtypeStruct((N, 4 * H), jnp.float32),
    )(p, g_prev, dinv, b.reshape(1, H), W)


def _final(p, g_prev, dinv, b, batch3, Wl, bl):
    def body(p_ref, g_ref, d_ref, b_ref, bat_ref, wl_ref, bl_ref,
             o_ref, acc_ref):
        i = pl.program_id(0)

        @pl.when(i == 0)
        def _():
            acc_ref[...] = jnp.zeros_like(acc_ref)

        dinv = d_ref[:, 0:1]
        h = dinv * (p_ref[0] + p_ref[1] + g_ref[:, :H]) + b_ref[...]
        bat = bat_ref[0]
        gid = lax.broadcasted_iota(jnp.int32, (G, R), 0)
        m = (gid == bat).astype(jnp.float32)
        haug = jnp.concatenate(
            [h, jnp.ones((R, H), jnp.float32)], axis=1)
        acc_ref[...] += jnp.dot(m, haug,
                                preferred_element_type=jnp.float32,
                                precision=lax.Precision.HIGHEST)

        @pl.when(i == N // R - 1)
        def _():
            sums = acc_ref[:, :H]
            cnt = acc_ref[:, H:]
            pooled = sums / jnp.maximum(cnt, 1.0)
            o_ref[...] = jnp.dot(pooled, wl_ref[...],
                                 preferred_element_type=jnp.float32,
                                 precision=lax.Precision.HIGHEST) + bl_ref[...]

    return pl.pallas_call(
        body,
        grid=(N // R,),
        in_specs=[
            pl.BlockSpec((NC, R, H), lambda i: (0, i, 0)),
            pl.BlockSpec((R, 2 * H), lambda i: (i, 0)),
            pl.BlockSpec((R, 8), lambda i: (i, 0)),
            pl.BlockSpec((1, H), lambda i: (0, 0)),
            pl.BlockSpec((1, 1, R), lambda i: (i, 0, 0)),
            pl.BlockSpec((H, C), lambda i: (0, 0)),
            pl.BlockSpec((1, C), lambda i: (0, 0)),
        ],
        out_specs=pl.BlockSpec((G, C), lambda i: (0, 0)),
        out_shape=jax.ShapeDtypeStruct((G, C), jnp.float32),
        scratch_shapes=[pltpu.VMEM((G, 2 * H), jnp.float32)],
    )(p, g_prev, dinv, b.reshape(1, H), batch3, Wl, bl.reshape(1, C))


def kernel(x, edge_index, batch, W1, b1, W2, b2, W3, b3, Wl, bl):
    src = edge_index[0].reshape(NW, EPW)
    dst = edge_index[1].reshape(NW, CH, K)
    batch3 = batch.reshape(N // R, 1, R)

    zeros_acc = jnp.zeros((NR, HP), jnp.float32)
    zeros_h = jnp.zeros((L, NHALF), jnp.float32)

    histp, gidx, aidx = _prep(src, dst, zeros_h)
    dinv = _degsum(histp)

    g1 = _mm_first(x, W1, dinv)
    p1 = _scat(g1.reshape(2 * N, HP), gidx, aidx, zeros_acc)
    g2 = _mm_mid(p1.reshape(NC, NPAD, H), g1, dinv, b1, W2)
    p2 = _scat(g2.reshape(2 * N, HP), gidx, aidx, zeros_acc)
    g3 = _mm_mid(p2.reshape(NC, NPAD, H), g2, dinv, b2, W3)
    p3 = _scat(g3.reshape(2 * N, HP), gidx, aidx, zeros_acc)
    return _final(p3.reshape(NC, NPAD, H), g3, dinv, b3, batch3, Wl, bl)
